# Initial kernel scaffold; baseline (speedup 1.0000x reference)
import jax
import jax.numpy as jnp
from jax.experimental import pallas as pl


def kernel(x, edge_index, W):
    raise NotImplementedError("write your pallas kernel here")


# speedup vs baseline: 4.7837x; 4.7837x over previous
import functools

import jax
import jax.numpy as jnp
from jax import lax
from jax.experimental import pallas as pl
from jax.experimental.pallas import tpu as pltpu
from jax.experimental.pallas import tpu_sc as plsc

NC = 2
NS = 16
NW = NC * NS
C = 128


def _sc_agg_kernel(n_pad, k, d, interpret=False):
    rps = n_pad // NS

    def body(x_hbm, src_hbm, dst_hbm, z_hbm, out_hbm,
             agg_sh, src_v, dst_v, gbuf, sem):
        cid = lax.axis_index("c")
        sid = lax.axis_index("s")
        wid = sid * NC + cid

        pltpu.sync_copy(z_hbm, agg_sh.at[pl.ds(sid * rps, rps)])
        pltpu.sync_copy(src_hbm.at[wid], src_v)
        pltpu.sync_copy(dst_hbm.at[wid], dst_v)
        plsc.subcore_barrier()

        def step(j, carry):
            pltpu.async_copy(x_hbm.at[src_v.at[j]], gbuf, sem).wait()
            pltpu.sync_copy(gbuf, agg_sh.at[dst_v.at[j]], add=True)
            return carry

        lax.fori_loop(0, k, step, 0)
        plsc.subcore_barrier()
        pltpu.sync_copy(agg_sh.at[pl.ds(sid * rps, rps)],
                        out_hbm.at[cid, pl.ds(sid * rps, rps)])

    mesh = plsc.VectorSubcoreMesh(core_axis_name="c", subcore_axis_name="s")
    return pl.kernel(
        body,
        out_type=jax.ShapeDtypeStruct((NC, n_pad, d), jnp.float32),
        mesh=mesh,
        scratch_types=[
            pltpu.VMEM_SHARED((n_pad, d), jnp.float32),
            pltpu.VMEM((k, C), jnp.int32),
            pltpu.VMEM((k, C), jnp.int32),
            pltpu.VMEM((C, d), jnp.float32),
            pltpu.SemaphoreType.DMA,
        ],
        interpret=interpret,
    )


def _tc_body(p0_ref, p1_ref, x_ref, wt_ref, o_ref):
    agg = p0_ref[...] + p1_ref[...]
    h = jnp.dot(agg, wt_ref[...], preferred_element_type=jnp.float32)
    o_ref[...] = jnp.maximum(h, 0.0) + x_ref[...]


@jax.jit
def kernel(x, edge_index, W):
    n, d = x.shape
    e = edge_index.shape[1]

    k = -(-e // (NW * C))
    e_pad = NW * k * C
    n_pad = -(-n // (NS * 8)) * (NS * 8)

    src = edge_index[0]
    dst = edge_index[1]
    src_p = jnp.concatenate(
        [src, jnp.zeros((e_pad - e,), jnp.int32)]).reshape(NW, k, C)
    dst_p = jnp.concatenate(
        [dst, jnp.full((e_pad - e,), n, jnp.int32)]).reshape(NW, k, C)
    zrows = jnp.zeros((n_pad // NS, d), jnp.float32)

    partials = _sc_agg_kernel(n_pad, k, d)(x, src_p, dst_p, zrows)

    nb = 8 * 125
    out = pl.pallas_call(
        _tc_body,
        out_shape=jax.ShapeDtypeStruct((n, d), jnp.float32),
        grid=(n // nb,),
        in_specs=[
            pl.BlockSpec((nb, d), lambda i: (i, 0)),
            pl.BlockSpec((nb, d), lambda i: (i, 0)),
            pl.BlockSpec((nb, d), lambda i: (i, 0)),
            pl.BlockSpec((d, d), lambda i: (0, 0)),
        ],
        out_specs=pl.BlockSpec((nb, d), lambda i: (i, 0)),
    )(partials[0, :n], partials[1, :n], x, W.T)
    return out
